# baseline (speedup 1.0000x reference)
import functools

import jax
import jax.numpy as jnp
from jax import lax
from jax.experimental import pallas as pl
from jax.experimental.pallas import tpu as pltpu
from jax.experimental.pallas import tpu_sc as plsc

D = 64
B = 4096
L = 200
NC, NS = 2, 16
NW = NC * NS
BPW = B // NW
IPW = BPW * L
G0, G1 = 128, 72
NLANE = 16
ND = D // NLANE


def _sc_pooled_sum(seq_flat, table):
    mesh = plsc.VectorSubcoreMesh(
        core_axis_name="c", subcore_axis_name="s",
        num_cores=NC, num_subcores=NS)

    @functools.partial(
        pl.kernel,
        out_type=jax.ShapeDtypeStruct((B, D), jnp.float32),
        mesh=mesh,
        compiler_params=pltpu.CompilerParams(use_tc_tiling_on_sc=False),
        scratch_types=[
            pltpu.VMEM((IPW,), jnp.int32),
            pltpu.VMEM((L, D), jnp.float32),
            pltpu.VMEM((BPW, D), jnp.float32),
            pltpu.SemaphoreType.DMA,
        ],
    )
    def k(seq_hbm, table_hbm, out_hbm, idx_v, rows_v, out_v, sem):
        wid = lax.axis_index("s") * NC + lax.axis_index("c")
        base = pl.multiple_of(wid * IPW, 8)
        pltpu.sync_copy(seq_hbm.at[pl.ds(base, IPW)], idx_v)

        def row_body(b, carry):
            off = pl.multiple_of(b * L, 8)
            cp0 = pltpu.async_copy(
                table_hbm.at[idx_v.at[pl.ds(off, G0)]],
                rows_v.at[pl.ds(0, G0), :], sem)
            cp1 = pltpu.async_copy(
                table_hbm.at[idx_v.at[pl.ds(off + G0, G1)]],
                rows_v.at[pl.ds(G0, G1), :], sem)
            cp0.wait()
            cp1.wait()

            def acc_body(j, accs):
                return tuple(accs[d] + rows_v[j, pl.ds(d * NLANE, NLANE)]
                             for d in range(ND))

            accs = tuple(jnp.zeros((NLANE,), jnp.float32) for _ in range(ND))
            accs = lax.fori_loop(0, L, acc_body, accs)
            for d in range(ND):
                out_v[b, pl.ds(d * NLANE, NLANE)] = accs[d]
            return carry

        lax.fori_loop(0, BPW, row_body, 0)
        rbase = pl.multiple_of(wid * BPW, 8)
        pltpu.sync_copy(out_v, out_hbm.at[pl.ds(rbase, BPW), :])

    return k(seq_flat, table)


def _tc_finalize(seq, sums, t0):
    bb = 256

    def body(seq_ref, sum_ref, t0_ref, out_ref):
        n0 = jnp.sum((seq_ref[...] == 0).astype(jnp.float32),
                     axis=1, keepdims=True)
        pooled = (sum_ref[...] - n0 * t0_ref[...]) / (jnp.float32(L) - n0 + 1e-9)
        nrm = jnp.sqrt(jnp.sum(pooled * pooled, axis=1, keepdims=True))
        out_ref[...] = pooled / jnp.maximum(nrm, 1e-12)

    return pl.pallas_call(
        body,
        grid=(B // bb,),
        in_specs=[
            pl.BlockSpec((bb, L), lambda i: (i, 0)),
            pl.BlockSpec((bb, D), lambda i: (i, 0)),
            pl.BlockSpec((1, D), lambda i: (0, 0)),
        ],
        out_specs=pl.BlockSpec((bb, D), lambda i: (i, 0)),
        out_shape=jax.ShapeDtypeStruct((B, D), jnp.float32),
    )(seq, sums, t0)


def kernel(seq, table):
    seq = seq.astype(jnp.int32)
    sums = _sc_pooled_sum(seq.reshape(-1), table)
    return _tc_finalize(seq, sums, table[0:1, :])


# speedup vs baseline: 1.2341x; 1.2341x over previous
import functools

import jax
import jax.numpy as jnp
from jax import lax
from jax.experimental import pallas as pl
from jax.experimental.pallas import tpu as pltpu
from jax.experimental.pallas import tpu_sc as plsc

D = 64
B = 4096
L = 200
NC, NS = 2, 16
NW = NC * NS
BPW = B // NW
IPW = BPW * L
G0, G1 = 128, 72
NLANE = 16
ND = D // NLANE


def _sc_pooled_sum(seq_flat, table):
    mesh = plsc.VectorSubcoreMesh(
        core_axis_name="c", subcore_axis_name="s",
        num_cores=NC, num_subcores=NS)

    nbuf = 4

    @functools.partial(
        pl.kernel,
        out_type=jax.ShapeDtypeStruct((B, D), jnp.float32),
        mesh=mesh,
        compiler_params=pltpu.CompilerParams(use_tc_tiling_on_sc=False),
        scratch_types=[
            pltpu.VMEM((IPW,), jnp.int32),
            [pltpu.VMEM((L, D), jnp.float32) for _ in range(nbuf)],
            pltpu.VMEM((BPW, D), jnp.float32),
            [pltpu.SemaphoreType.DMA for _ in range(nbuf)],
        ],
    )
    def k(seq_hbm, table_hbm, out_hbm, idx_v, rows_bufs, out_v, sems):
        wid = lax.axis_index("s") * NC + lax.axis_index("c")
        base = pl.multiple_of(wid * IPW, 8)
        pltpu.sync_copy(seq_hbm.at[pl.ds(base, IPW)], idx_v)

        def issue(p, b):
            off = pl.multiple_of(b * L, 8)
            pltpu.async_copy(
                table_hbm.at[idx_v.at[pl.ds(off, G0)]],
                rows_bufs[p].at[pl.ds(0, G0), :], sems[p])
            pltpu.async_copy(
                table_hbm.at[idx_v.at[pl.ds(off + G0, G1)]],
                rows_bufs[p].at[pl.ds(G0, G1), :], sems[p])

        def drain(p):
            pltpu.make_async_copy(
                table_hbm.at[idx_v.at[pl.ds(0, G0)]],
                rows_bufs[p].at[pl.ds(0, G0), :], sems[p]).wait()
            pltpu.make_async_copy(
                table_hbm.at[idx_v.at[pl.ds(0, G1)]],
                rows_bufs[p].at[pl.ds(G0, G1), :], sems[p]).wait()

        def reduce_into(p, b):
            rows_v = rows_bufs[p]

            def acc_body(j, accs):
                return tuple(accs[d] + rows_v[j, pl.ds(d * NLANE, NLANE)]
                             for d in range(ND))

            accs = tuple(jnp.zeros((NLANE,), jnp.float32) for _ in range(ND))
            accs = lax.fori_loop(0, L, acc_body, accs, unroll=8)
            for d in range(ND):
                out_v[b, pl.ds(d * NLANE, NLANE)] = accs[d]

        for p in range(nbuf):
            issue(p, p)

        def group_body(g, carry):
            for p in range(nbuf):
                b = g * nbuf + p
                drain(p)

                @pl.when(b + nbuf < BPW)
                def _():
                    issue(p, b + nbuf)

                reduce_into(p, b)
            return carry

        lax.fori_loop(0, BPW // nbuf, group_body, 0)
        rbase = pl.multiple_of(wid * BPW, 8)
        pltpu.sync_copy(out_v, out_hbm.at[pl.ds(rbase, BPW), :])

    return k(seq_flat, table)


def _tc_finalize(seq, sums, t0):
    bb = 256

    def body(seq_ref, sum_ref, t0_ref, out_ref):
        n0 = jnp.sum((seq_ref[...] == 0).astype(jnp.float32),
                     axis=1, keepdims=True)
        pooled = (sum_ref[...] - n0 * t0_ref[...]) / (jnp.float32(L) - n0 + 1e-9)
        nrm = jnp.sqrt(jnp.sum(pooled * pooled, axis=1, keepdims=True))
        out_ref[...] = pooled / jnp.maximum(nrm, 1e-12)

    return pl.pallas_call(
        body,
        grid=(B // bb,),
        in_specs=[
            pl.BlockSpec((bb, L), lambda i: (i, 0)),
            pl.BlockSpec((bb, D), lambda i: (i, 0)),
            pl.BlockSpec((1, D), lambda i: (0, 0)),
        ],
        out_specs=pl.BlockSpec((bb, D), lambda i: (i, 0)),
        out_shape=jax.ShapeDtypeStruct((B, D), jnp.float32),
    )(seq, sums, t0)


def kernel(seq, table):
    seq = seq.astype(jnp.int32)
    sums = _sc_pooled_sum(seq.reshape(-1), table)
    return _tc_finalize(seq, sums, table[0:1, :])
